# baseline (speedup 1.0000x reference)
import functools

import jax
import jax.numpy as jnp
from jax import lax
from jax.experimental import pallas as pl
from jax.experimental.pallas import tpu as pltpu
from jax.experimental.pallas import tpu_sc as plsc

N = 10000
E = 320000
D_IN = 128
H = 256
C = 40
G = 64
NP = 10240

NSC = 2
NSUB = 16

CH = 128

EPT16 = E // NSUB
EPT32 = E // (NSC * NSUB)

_MESH = dict(core_axis_name="c", subcore_axis_name="s")


def _zero_vec():
    return jnp.zeros((16,), jnp.float32)


import numpy as _np
_BCAST = [_np.full((16,), i, _np.int32) for i in range(16)]


def _lane_bcast(cv, e16):
    idx = lax.iota(jnp.int32, 16) * 0 + e16
    return lax.gather(
        cv, idx[:, None],
        dimension_numbers=lax.GatherDimensionNumbers(
            offset_dims=(), collapsed_slice_dims=(0,), start_index_map=(0,)),
        slice_sizes=(1,), mode=lax.GatherScatterMode.PROMISE_IN_BOUNDS)

def _fisr(d):
    i = lax.bitcast_convert_type(d, jnp.int32)
    y = lax.bitcast_convert_type(
        jnp.int32(0x5F3759DF) - lax.shift_right_logical(i, 1), jnp.float32)
    for _ in range(4):
        y = y * (1.5 - 0.5 * d * y * y)
    return y


def _prep_body(src_hbm, dst_hbm, ew_hbm, batch_hbm,
               d2_hbm, c_hbm, p0_hbm, p1_hbm,
               deg_sp, dis_sp, p_sp,
               zvm, dvm, svm, evm, cvm, pvm, dsb, ddb, bbuf,
               d32, e32, s16, d16, e16,
               disbuf, d2buf, sem):
    cid = lax.axis_index("c")
    sid = lax.axis_index("s")
    wid = sid * NSC + cid

    def z_body(i, _):
        zvm[pl.ds(i * 16, 16)] = _zero_vec()
        return 0
    lax.fori_loop(0, 160, z_body, 0)
    pltpu.sync_copy(zvm.at[pl.ds(0, 640)], deg_sp.at[pl.ds(sid * 640, 640)])
    for k in range(16):
        pltpu.sync_copy(zvm, p_sp.at[pl.ds(sid * 40960 + k * 2560, 2560)])
    plsc.subcore_barrier()

    base_deg = sid * EPT16

    def deg_chunk(i, _):
        off = base_deg + i * CH
        pltpu.sync_copy(dst_hbm.at[pl.ds(off, CH)], dvm)
        pltpu.sync_copy(ew_hbm.at[pl.ds(off, CH)], evm)
        pltpu.sync_copy(evm, deg_sp.at[dvm], add=True)
        return 0
    nfull = EPT16 // CH
    lax.fori_loop(0, nfull, deg_chunk, 0)
    rem = EPT16 - nfull * CH
    off = base_deg + nfull * CH
    pltpu.sync_copy(dst_hbm.at[pl.ds(off, rem)], d32)
    pltpu.sync_copy(ew_hbm.at[pl.ds(off, rem)], e32)
    pltpu.sync_copy(e32, deg_sp.at[d32], add=True)
    plsc.subcore_barrier()

    pltpu.sync_copy(deg_sp.at[pl.ds(sid * 640, 640)], disbuf)

    def dis_body(i, _):
        d = disbuf[pl.ds(i * 16, 16)] + 1.0
        y = _fisr(d)
        disbuf[pl.ds(i * 16, 16)] = y
        d2buf[pl.ds(i * 16, 16)] = y * y
        return 0
    lax.fori_loop(0, 40, dis_body, 0)
    pltpu.sync_copy(disbuf, dis_sp.at[pl.ds(sid * 640, 640)])

    @pl.when(cid == 0)
    def _():
        pltpu.sync_copy(d2buf, d2_hbm.at[pl.ds(sid * 640, 640)])
    plsc.subcore_barrier()

    base_c = wid * EPT32

    def c_chunk(i, _):
        off = base_c + i * CH
        pltpu.sync_copy(src_hbm.at[pl.ds(off, CH)], svm)
        pltpu.sync_copy(dst_hbm.at[pl.ds(off, CH)], dvm)
        pltpu.sync_copy(ew_hbm.at[pl.ds(off, CH)], evm)
        pltpu.async_copy(dis_sp.at[svm], dsb, sem).wait()
        pltpu.async_copy(dis_sp.at[dvm], ddb, sem).wait()
        pltpu.async_copy(batch_hbm.at[dvm], bbuf, sem).wait()

        def inner(j, _):
            sl = pl.ds(j * 16, 16)
            cvm[sl] = evm[sl] * dsb[sl] * ddb[sl]
            pvm[sl] = bbuf[sl] * NP + svm[sl]
            return 0
        lax.fori_loop(0, CH // 16, inner, 0)
        pltpu.sync_copy(cvm, c_hbm.at[pl.ds(off, CH)])
        pltpu.sync_copy(cvm, p_sp.at[pvm], add=True)
        return 0
    nfull_c = EPT32 // CH
    lax.fori_loop(0, nfull_c, c_chunk, 0)
    rem_c = EPT32 - nfull_c * CH
    off = base_c + nfull_c * CH
    pltpu.sync_copy(src_hbm.at[pl.ds(off, rem_c)], s16)
    pltpu.sync_copy(dst_hbm.at[pl.ds(off, rem_c)], d16)
    pltpu.sync_copy(ew_hbm.at[pl.ds(off, rem_c)], e16)
    pltpu.async_copy(dis_sp.at[s16], dsb.at[pl.ds(0, 16)], sem).wait()
    pltpu.async_copy(dis_sp.at[d16], ddb.at[pl.ds(0, 16)], sem).wait()
    pltpu.async_copy(batch_hbm.at[d16], bbuf.at[pl.ds(0, 16)], sem).wait()
    sl16 = pl.ds(0, 16)
    cvm[sl16] = e16[...] * dsb[sl16] * ddb[sl16]
    pvm16 = bbuf[sl16] * NP + s16[...]
    pltpu.sync_copy(cvm.at[sl16], c_hbm.at[pl.ds(off, rem_c)])
    pltpu.sync_copy(cvm.at[sl16], p_sp.at[pvm16], add=True)
    plsc.subcore_barrier()

    @pl.when(cid == 0)
    def _():
        pltpu.sync_copy(p_sp.at[pl.ds(sid * 40960, 40960)],
                        p0_hbm.at[pl.ds(sid * 40960, 40960)])

    @pl.when(cid == 1)
    def _():
        pltpu.sync_copy(p_sp.at[pl.ds(sid * 40960, 40960)],
                        p1_hbm.at[pl.ds(sid * 40960, 40960)])


def _make_prep():
    f32, i32 = jnp.float32, jnp.int32
    return pl.kernel(
        _prep_body,
        out_type=(
            jax.ShapeDtypeStruct((NP,), f32),
            jax.ShapeDtypeStruct((E,), f32),
            jax.ShapeDtypeStruct((G * NP,), f32),
            jax.ShapeDtypeStruct((G * NP,), f32),
        ),
        mesh=plsc.VectorSubcoreMesh(**_MESH),
        scratch_types=[
            pltpu.VMEM_SHARED((NP,), f32),
            pltpu.VMEM_SHARED((NP,), f32),
            pltpu.VMEM_SHARED((G * NP,), f32),
            pltpu.VMEM((2560,), f32),
            pltpu.VMEM((CH,), i32),
            pltpu.VMEM((CH,), i32),
            pltpu.VMEM((CH,), f32),
            pltpu.VMEM((CH,), f32),
            pltpu.VMEM((CH,), i32),
            pltpu.VMEM((CH,), f32),
            pltpu.VMEM((CH,), f32),
            pltpu.VMEM((CH,), i32),
            pltpu.VMEM((32,), i32),
            pltpu.VMEM((32,), f32),
            pltpu.VMEM((16,), i32),
            pltpu.VMEM((16,), i32),
            pltpu.VMEM((16,), f32),
            pltpu.VMEM((640,), f32),
            pltpu.VMEM((640,), f32),
            pltpu.SemaphoreType.DMA,
        ],
        name="gcn_prep_sc",
    )


CH64 = 64
ERR = 5120
CPT = ERR // 32
E_PAD = ERR * CH64


def _prop_rounds(h_list, out_list, cid, sid, acc_sp,
                 sd3, sdr, rows, scaled, gsem, ssem):
    wid = sid * NSC + cid
    gbase = wid * CPT

    def fire_gather(h_hbm, ps, b):
        return pltpu.async_copy(h_hbm.at[sdr.at[ps, b, 0]], rows[b], gsem[b])

    def wait_gather(h_hbm, ps, b):
        pltpu.make_async_copy(h_hbm.at[sdr.at[ps, b, 0]], rows[b],
                              gsem[b]).wait()

    def fire_scatter(ps, b):
        return pltpu.async_copy(scaled[b], acc_sp.at[sdr.at[ps, b, 1]],
                                ssem[b], add=True)

    def wait_scatter(ps, b):
        pltpu.make_async_copy(scaled[b], acc_sp.at[sdr.at[ps, b, 1]],
                              ssem[b]).wait()

    def copy_sd(pr, ps):
        pltpu.sync_copy(sd3.at[pl.ds(gbase + 2 * pr, 2)], sdr.at[ps])

    def scale(ps, b):
        def sj(j, _):
            cv = lax.bitcast_convert_type(
                sdr[ps, b, 2, pl.ds(j * 16, 16)], jnp.float32)
            for e16 in range(16):
                e = j * 16 + e16
                cb = _lane_bcast(cv, e16)
                for k in range(8):
                    scaled[b][e, pl.ds(k * 16, 16)] = (
                        rows[b][e, pl.ds(k * 16, 16)] * cb)
            return 0
        lax.fori_loop(0, CH64 // 16, sj, 0)

    for r, (h_hbm, (o0_hbm, o1_hbm)) in enumerate(zip(h_list, out_list)):
        def z_body(i, _):
            scaled[0][i // 8, pl.ds((i % 8) * 16, 16)] = _zero_vec()
            return 0
        lax.fori_loop(0, CH64 * 8, z_body, 0)
        for k in range(10):
            pltpu.sync_copy(scaled[0],
                            acc_sp.at[pl.ds(sid * 640 + k * 64, 64)])
        plsc.subcore_barrier()

        copy_sd(0, 0)
        copy_sd(1, 1)
        fire_gather(h_hbm, 0, 0)
        fire_gather(h_hbm, 0, 1)

        def q_body(q, _):
            for bq in range(2):
                pr = 2 * q + bq
                ps = bq
                for b in range(2):
                    wait_gather(h_hbm, ps, b)

                    @pl.when(pr >= 1)
                    def _():
                        wait_scatter(ps, b)
                    scale(ps, b)
                    fire_scatter(ps, b)

                    @pl.when(pr + 1 < CPT // 2)
                    def _():
                        fire_gather(h_hbm, 1 - ps, b)

                @pl.when(pr + 2 < CPT // 2)
                def _():
                    copy_sd(pr + 2, ps)
            return 0
        lax.fori_loop(0, CPT // 4, q_body, 0)
        wait_scatter(1, 0)
        wait_scatter(1, 1)
        plsc.subcore_barrier()

        @pl.when(cid == 0)
        def _():
            pltpu.sync_copy(acc_sp.at[pl.ds(sid * 640, 640)],
                            o0_hbm.at[pl.ds(sid * 640, 640)])

        @pl.when(cid == 1)
        def _():
            pltpu.sync_copy(acc_sp.at[pl.ds(sid * 640, 640)],
                            o1_hbm.at[pl.ds(sid * 640, 640)])
        if r + 1 < len(h_list):
            plsc.subcore_barrier()


def _prop1_body(h_hbm, sd3, o0, o1,
                acc_sp, sdr, rows0, rows1, sc0, sc1,
                gs0, gs1, ss0, ss1):
    cid = lax.axis_index("c")
    sid = lax.axis_index("s")
    _prop_rounds([h_hbm], [(o0, o1)], cid, sid, acc_sp, sd3,
                 sdr, (rows0, rows1), (sc0, sc1),
                 (gs0, gs1), (ss0, ss1))


def _prop2_body(ha_hbm, hb_hbm, sd3, oa0, oa1, ob0, ob1,
                acc_sp, sdr, rows0, rows1, sc0, sc1,
                gs0, gs1, ss0, ss1):
    cid = lax.axis_index("c")
    sid = lax.axis_index("s")
    _prop_rounds([ha_hbm, hb_hbm], [(oa0, oa1), (ob0, ob1)],
                 cid, sid, acc_sp, sd3,
                 sdr, (rows0, rows1), (sc0, sc1),
                 (gs0, gs1), (ss0, ss1))


def _prop_scratch():
    f32, i32 = jnp.float32, jnp.int32
    return [
        pltpu.VMEM_SHARED((NP, 128), f32),
        pltpu.VMEM((2, 2, 3, CH64), i32),
        pltpu.VMEM((CH64, 128), f32),
        pltpu.VMEM((CH64, 128), f32),
        pltpu.VMEM((CH64, 128), f32),
        pltpu.VMEM((CH64, 128), f32),
        pltpu.SemaphoreType.DMA,
        pltpu.SemaphoreType.DMA,
        pltpu.SemaphoreType.DMA,
        pltpu.SemaphoreType.DMA,
    ]


def _make_prop1():
    f32 = jnp.float32
    return pl.kernel(
        _prop1_body,
        out_type=(
            jax.ShapeDtypeStruct((NP, 128), f32),
            jax.ShapeDtypeStruct((NP, 128), f32),
        ),
        mesh=plsc.VectorSubcoreMesh(**_MESH),
        scratch_types=_prop_scratch(),
        name="gcn_prop1_sc",
    )


def _make_prop2():
    f32 = jnp.float32
    return pl.kernel(
        _prop2_body,
        out_type=tuple(
            jax.ShapeDtypeStruct((NP, 128), f32) for _ in range(4)),
        mesh=plsc.VectorSubcoreMesh(**_MESH),
        scratch_types=_prop_scratch(),
        name="gcn_prop2_sc",
    )


def _lrelu(v):
    return jnp.where(v >= 0, v, 0.01 * v)


def _t2_body(e1p0, e1p1, x, d2, w1, b1, w2, qa, qb):
    z = e1p0[...] + e1p1[...] + d2[...] * x[...]
    h1 = jnp.dot(z, w1[...], preferred_element_type=jnp.float32) + b1[...]
    h1 = _lrelu(h1)
    q = jnp.dot(h1, w2[...], preferred_element_type=jnp.float32)
    qa[...] = q[:, :128]
    qb[...] = q[:, 128:]


def _make_t2():
    f32 = jnp.float32
    R = 256
    grid = (NP // R,)
    return pl.pallas_call(
        _t2_body,
        grid=grid,
        in_specs=[
            pl.BlockSpec((R, 128), lambda t: (t, 0)),
            pl.BlockSpec((R, 128), lambda t: (t, 0)),
            pl.BlockSpec((R, 128), lambda t: (t, 0)),
            pl.BlockSpec((R, 1), lambda t: (t, 0)),
            pl.BlockSpec((128, 256), lambda t: (0, 0)),
            pl.BlockSpec((1, 256), lambda t: (0, 0)),
            pl.BlockSpec((256, 256), lambda t: (0, 0)),
        ],
        out_specs=[
            pl.BlockSpec((R, 128), lambda t: (t, 0)),
            pl.BlockSpec((R, 128), lambda t: (t, 0)),
        ],
        out_shape=[
            jax.ShapeDtypeStruct((NP, 128), f32),
            jax.ShapeDtypeStruct((NP, 128), f32),
        ],
    )


def _t3_body(e2a0, e2a1, e2b0, e2b1, qa, qb, d2c, b2, bat, d2r, p0, p1,
             w3, b3, fw1, fb1, fw2, fb2, fw3, fb3,
             out, psum, cnt):
    t = pl.program_id(0)
    nt = pl.num_programs(0)

    @pl.when(t == 0)
    def _():
        psum[...] = jnp.zeros_like(psum)
        cnt[...] = jnp.zeros_like(cnt)

    d2v = d2c[...]
    z = jnp.concatenate([e2a0[...] + e2a1[...] + d2v * qa[...],
                         e2b0[...] + e2b1[...] + d2v * qb[...]],
                        axis=1) + b2[...]
    h2 = _lrelu(z)
    g = lax.broadcasted_iota(jnp.int32, (G, 256), 0)
    cmp = bat[...] == g
    mt = p0[...] + p1[...] + jnp.where(cmp, d2r[...], 0.0)
    psum[...] += jnp.dot(mt, h2, preferred_element_type=jnp.float32)
    cnt[:, 0:1] += jnp.sum(cmp.astype(jnp.float32), axis=1, keepdims=True)

    @pl.when(t == nt - 1)
    def _():
        cg = cnt[:, 0:1]
        pooled = psum[...] / jnp.maximum(cg, 1.0)
        h3 = jnp.dot(pooled, w3[...], preferred_element_type=jnp.float32)
        h3 = h3 + jnp.where(cg > 0, b3[...], 0.0)
        z1 = _lrelu(jnp.dot(h3, fw1[...],
                            preferred_element_type=jnp.float32) + fb1[...])
        z2 = _lrelu(jnp.dot(z1, fw2[...],
                            preferred_element_type=jnp.float32) + fb2[...])
        out[...] = jnp.dot(z2, fw3[...],
                           preferred_element_type=jnp.float32) + fb3[...]


def _make_t3():
    f32 = jnp.float32
    R = 256
    grid = (NP // R,)
    return pl.pallas_call(
        _t3_body,
        grid=grid,
        in_specs=[
            pl.BlockSpec((R, 128), lambda t: (t, 0)),
            pl.BlockSpec((R, 128), lambda t: (t, 0)),
            pl.BlockSpec((R, 128), lambda t: (t, 0)),
            pl.BlockSpec((R, 128), lambda t: (t, 0)),
            pl.BlockSpec((R, 128), lambda t: (t, 0)),
            pl.BlockSpec((R, 128), lambda t: (t, 0)),
            pl.BlockSpec((R, 1), lambda t: (t, 0)),
            pl.BlockSpec((1, 256), lambda t: (0, 0)),
            pl.BlockSpec((1, R), lambda t: (0, t)),
            pl.BlockSpec((1, R), lambda t: (0, t)),
            pl.BlockSpec((G, R), lambda t: (0, t)),
            pl.BlockSpec((G, R), lambda t: (0, t)),
            pl.BlockSpec((256, 256), lambda t: (0, 0)),
            pl.BlockSpec((1, 256), lambda t: (0, 0)),
            pl.BlockSpec((256, 128), lambda t: (0, 0)),
            pl.BlockSpec((1, 128), lambda t: (0, 0)),
            pl.BlockSpec((128, 64), lambda t: (0, 0)),
            pl.BlockSpec((1, 64), lambda t: (0, 0)),
            pl.BlockSpec((64, C), lambda t: (0, 0)),
            pl.BlockSpec((1, C), lambda t: (0, 0)),
        ],
        out_specs=pl.BlockSpec((G, C), lambda t: (0, 0)),
        out_shape=jax.ShapeDtypeStruct((G, C), f32),
        scratch_shapes=[
            pltpu.VMEM((G, 256), f32),
            pltpu.VMEM((G, 128), f32),
        ],
    )


_prep = _make_prep()
_prop1 = _make_prop1()
_prop2 = _make_prop2()
_t2 = _make_t2()
_t3 = _make_t3()


def kernel(x, edge_index, edge_weight, batch,
           W1, b1, W2, b2, W3, b3, FW1, Fb1, FW2, Fb2, FW3, Fb3):
    f32 = jnp.float32
    src = edge_index[0]
    dst = edge_index[1]
    x_pad = jnp.pad(x, ((0, NP - N), (0, 0)))
    batch_pad = jnp.pad(batch, (0, NP - N), constant_values=-1)

    d2, c, p0, p1 = _prep(src, dst, edge_weight, batch_pad)

    src2d = jnp.pad(src, (0, E_PAD - E)).reshape(ERR, CH64)
    dst2d = jnp.pad(dst, (0, E_PAD - E)).reshape(ERR, CH64)
    c2d = jnp.pad(c, (0, E_PAD - E)).reshape(ERR, CH64)
    sd3 = jnp.stack(
        [src2d, dst2d, lax.bitcast_convert_type(c2d, jnp.int32)], axis=1)

    e1p0, e1p1 = _prop1(x_pad, sd3)

    d2c = d2.reshape(NP, 1)
    qa, qb = _t2(e1p0, e1p1, x_pad, d2c, W1, b1.reshape(1, H), W2)

    e2a0, e2a1, e2b0, e2b1 = _prop2(qa, qb, sd3)

    out = _t3(e2a0, e2a1, e2b0, e2b1, qa, qb, d2c, b2.reshape(1, H),
              batch_pad.reshape(1, NP), d2.reshape(1, NP),
              p0.reshape(G, NP), p1.reshape(G, NP),
              W3, b3.reshape(1, H), FW1, Fb1.reshape(1, H // 2),
              FW2, Fb2.reshape(1, H // 4), FW3, Fb3.reshape(1, C))
    return out


# speedup vs baseline: 1.4779x; 1.4779x over previous
import functools

import jax
import jax.numpy as jnp
from jax import lax
from jax.experimental import pallas as pl
from jax.experimental.pallas import tpu as pltpu
from jax.experimental.pallas import tpu_sc as plsc

N = 10000
E = 320000
D_IN = 128
H = 256
C = 40
G = 64
NP = 10240

NSC = 2
NSUB = 16

CH = 128

EPT16 = E // NSUB
EPT32 = E // (NSC * NSUB)

_MESH = dict(core_axis_name="c", subcore_axis_name="s")


def _zero_vec():
    return jnp.zeros((16,), jnp.float32)


import numpy as _np
_BCAST = [_np.full((16,), i, _np.int32) for i in range(16)]


def _lane_bcast(cv, e16):
    idx = lax.iota(jnp.int32, 16) * 0 + e16
    return lax.gather(
        cv, idx[:, None],
        dimension_numbers=lax.GatherDimensionNumbers(
            offset_dims=(), collapsed_slice_dims=(0,), start_index_map=(0,)),
        slice_sizes=(1,), mode=lax.GatherScatterMode.PROMISE_IN_BOUNDS)

def _fisr(d):
    i = lax.bitcast_convert_type(d, jnp.int32)
    y = lax.bitcast_convert_type(
        jnp.int32(0x5F3759DF) - lax.shift_right_logical(i, 1), jnp.float32)
    for _ in range(4):
        y = y * (1.5 - 0.5 * d * y * y)
    return y


def _prep_body(src_hbm, dst_hbm, ew_hbm, batch_hbm,
               d2_hbm, c_hbm, p0_hbm, p1_hbm,
               deg_sp, dis_sp, p_sp,
               zvm, dvm, svm, evm, cvm, pvm, dsb, ddb, bbuf,
               d32, e32, s16, d16, e16,
               disbuf, d2buf, sem):
    cid = lax.axis_index("c")
    sid = lax.axis_index("s")
    wid = sid * NSC + cid

    def z_body(i, _):
        zvm[pl.ds(i * 16, 16)] = _zero_vec()
        return 0
    lax.fori_loop(0, 160, z_body, 0)
    pltpu.sync_copy(zvm.at[pl.ds(0, 640)], deg_sp.at[pl.ds(sid * 640, 640)])
    for k in range(16):
        pltpu.sync_copy(zvm, p_sp.at[pl.ds(sid * 40960 + k * 2560, 2560)])
    plsc.subcore_barrier()

    base_deg = sid * EPT16

    def deg_chunk(i, _):
        off = base_deg + i * CH
        pltpu.sync_copy(dst_hbm.at[pl.ds(off, CH)], dvm)
        pltpu.sync_copy(ew_hbm.at[pl.ds(off, CH)], evm)
        pltpu.sync_copy(evm, deg_sp.at[dvm], add=True)
        return 0
    nfull = EPT16 // CH
    lax.fori_loop(0, nfull, deg_chunk, 0)
    rem = EPT16 - nfull * CH
    off = base_deg + nfull * CH
    pltpu.sync_copy(dst_hbm.at[pl.ds(off, rem)], d32)
    pltpu.sync_copy(ew_hbm.at[pl.ds(off, rem)], e32)
    pltpu.sync_copy(e32, deg_sp.at[d32], add=True)
    plsc.subcore_barrier()

    pltpu.sync_copy(deg_sp.at[pl.ds(sid * 640, 640)], disbuf)

    def dis_body(i, _):
        d = disbuf[pl.ds(i * 16, 16)] + 1.0
        y = _fisr(d)
        disbuf[pl.ds(i * 16, 16)] = y
        d2buf[pl.ds(i * 16, 16)] = y * y
        return 0
    lax.fori_loop(0, 40, dis_body, 0)
    pltpu.sync_copy(disbuf, dis_sp.at[pl.ds(sid * 640, 640)])

    @pl.when(cid == 0)
    def _():
        pltpu.sync_copy(d2buf, d2_hbm.at[pl.ds(sid * 640, 640)])
    plsc.subcore_barrier()

    base_c = wid * EPT32

    def c_chunk(i, _):
        off = base_c + i * CH
        pltpu.sync_copy(src_hbm.at[pl.ds(off, CH)], svm)
        pltpu.sync_copy(dst_hbm.at[pl.ds(off, CH)], dvm)
        pltpu.sync_copy(ew_hbm.at[pl.ds(off, CH)], evm)
        pltpu.async_copy(dis_sp.at[svm], dsb, sem).wait()
        pltpu.async_copy(dis_sp.at[dvm], ddb, sem).wait()
        pltpu.async_copy(batch_hbm.at[dvm], bbuf, sem).wait()

        def inner(j, _):
            sl = pl.ds(j * 16, 16)
            cvm[sl] = evm[sl] * dsb[sl] * ddb[sl]
            pvm[sl] = bbuf[sl] * NP + svm[sl]
            return 0
        lax.fori_loop(0, CH // 16, inner, 0)
        pltpu.sync_copy(cvm, c_hbm.at[pl.ds(off, CH)])
        pltpu.sync_copy(cvm, p_sp.at[pvm], add=True)
        return 0
    nfull_c = EPT32 // CH
    lax.fori_loop(0, nfull_c, c_chunk, 0)
    rem_c = EPT32 - nfull_c * CH
    off = base_c + nfull_c * CH
    pltpu.sync_copy(src_hbm.at[pl.ds(off, rem_c)], s16)
    pltpu.sync_copy(dst_hbm.at[pl.ds(off, rem_c)], d16)
    pltpu.sync_copy(ew_hbm.at[pl.ds(off, rem_c)], e16)
    pltpu.async_copy(dis_sp.at[s16], dsb.at[pl.ds(0, 16)], sem).wait()
    pltpu.async_copy(dis_sp.at[d16], ddb.at[pl.ds(0, 16)], sem).wait()
    pltpu.async_copy(batch_hbm.at[d16], bbuf.at[pl.ds(0, 16)], sem).wait()
    sl16 = pl.ds(0, 16)
    cvm[sl16] = e16[...] * dsb[sl16] * ddb[sl16]
    pvm16 = bbuf[sl16] * NP + s16[...]
    pltpu.sync_copy(cvm.at[sl16], c_hbm.at[pl.ds(off, rem_c)])
    pltpu.sync_copy(cvm.at[sl16], p_sp.at[pvm16], add=True)
    plsc.subcore_barrier()

    @pl.when(cid == 0)
    def _():
        pltpu.sync_copy(p_sp.at[pl.ds(sid * 40960, 40960)],
                        p0_hbm.at[pl.ds(sid * 40960, 40960)])

    @pl.when(cid == 1)
    def _():
        pltpu.sync_copy(p_sp.at[pl.ds(sid * 40960, 40960)],
                        p1_hbm.at[pl.ds(sid * 40960, 40960)])


def _make_prep():
    f32, i32 = jnp.float32, jnp.int32
    return pl.kernel(
        _prep_body,
        out_type=(
            jax.ShapeDtypeStruct((NP,), f32),
            jax.ShapeDtypeStruct((E,), f32),
            jax.ShapeDtypeStruct((G * NP,), f32),
            jax.ShapeDtypeStruct((G * NP,), f32),
        ),
        mesh=plsc.VectorSubcoreMesh(**_MESH),
        scratch_types=[
            pltpu.VMEM_SHARED((NP,), f32),
            pltpu.VMEM_SHARED((NP,), f32),
            pltpu.VMEM_SHARED((G * NP,), f32),
            pltpu.VMEM((2560,), f32),
            pltpu.VMEM((CH,), i32),
            pltpu.VMEM((CH,), i32),
            pltpu.VMEM((CH,), f32),
            pltpu.VMEM((CH,), f32),
            pltpu.VMEM((CH,), i32),
            pltpu.VMEM((CH,), f32),
            pltpu.VMEM((CH,), f32),
            pltpu.VMEM((CH,), i32),
            pltpu.VMEM((32,), i32),
            pltpu.VMEM((32,), f32),
            pltpu.VMEM((16,), i32),
            pltpu.VMEM((16,), i32),
            pltpu.VMEM((16,), f32),
            pltpu.VMEM((640,), f32),
            pltpu.VMEM((640,), f32),
            pltpu.SemaphoreType.DMA,
        ],
        name="gcn_prep_sc",
    )


CH64 = 64
ERR = 5120
CPT = ERR // 32
E_PAD = ERR * CH64


def _prop_rounds(h_list, out_list, cid, sid, acc_sp,
                 sd3, sdr, rows, scaled, gsem, ssem):
    wid = sid * NSC + cid
    gbase = wid * CPT

    def fire_gather(h_hbm, ps, b):
        return pltpu.async_copy(h_hbm.at[sdr.at[ps, b, 0]], rows[b], gsem[b])

    def wait_gather(h_hbm, ps, b):
        pltpu.make_async_copy(h_hbm.at[sdr.at[ps, b, 0]], rows[b],
                              gsem[b]).wait()

    def fire_scatter(ps, b):
        return pltpu.async_copy(scaled[b], acc_sp.at[sdr.at[ps, b, 1]],
                                ssem[b], add=True)

    def wait_scatter(ps, b):
        pltpu.make_async_copy(scaled[b], acc_sp.at[sdr.at[ps, b, 1]],
                              ssem[b]).wait()

    def copy_sd(pr, ps):
        pltpu.sync_copy(sd3.at[pl.ds(gbase + 2 * pr, 2)], sdr.at[ps])

    def scale(ps, b):
        def sj(j, _):
            cv = lax.bitcast_convert_type(
                sdr[ps, b, 2, pl.ds(j * 16, 16)], jnp.float32)
            for e16 in range(16):
                e = j * 16 + e16
                cb = _lane_bcast(cv, e16)
                for k in range(8):
                    scaled[b][e, pl.ds(k * 16, 16)] = (
                        rows[b][e, pl.ds(k * 16, 16)] * cb)
            return 0
        lax.fori_loop(0, CH64 // 16, sj, 0)

    for r, (h_hbm, (o0_hbm, o1_hbm)) in enumerate(zip(h_list, out_list)):
        def z_body(i, _):
            scaled[0][i // 8, pl.ds((i % 8) * 16, 16)] = _zero_vec()
            return 0
        lax.fori_loop(0, CH64 * 8, z_body, 0)
        for k in range(10):
            pltpu.sync_copy(scaled[0],
                            acc_sp.at[pl.ds(sid * 640 + k * 64, 64)])
        plsc.subcore_barrier()

        copy_sd(0, 0)
        copy_sd(1, 1)
        fire_gather(h_hbm, 0, 0)
        fire_gather(h_hbm, 0, 1)

        def q_body(q, _):
            for bq in range(2):
                pr = 2 * q + bq
                ps = bq
                for b in range(2):
                    wait_gather(h_hbm, ps, b)

                    @pl.when(pr >= 1)
                    def _():
                        wait_scatter(ps, b)
                    scale(ps, b)
                    fire_scatter(ps, b)

                    @pl.when(pr + 1 < CPT // 2)
                    def _():
                        fire_gather(h_hbm, 1 - ps, b)

                @pl.when(pr + 2 < CPT // 2)
                def _():
                    copy_sd(pr + 2, ps)
            return 0
        lax.fori_loop(0, CPT // 4, q_body, 0)
        wait_scatter(1, 0)
        wait_scatter(1, 1)
        plsc.subcore_barrier()

        @pl.when(cid == 0)
        def _():
            pltpu.sync_copy(acc_sp.at[pl.ds(sid * 640, 640)],
                            o0_hbm.at[pl.ds(sid * 640, 640)])

        @pl.when(cid == 1)
        def _():
            pltpu.sync_copy(acc_sp.at[pl.ds(sid * 640, 640)],
                            o1_hbm.at[pl.ds(sid * 640, 640)])
        if r + 1 < len(h_list):
            plsc.subcore_barrier()


def _prop1_body(h_hbm, sd3, o0, o1,
                acc_sp, sdr, rows0, rows1, sc0, sc1,
                gs0, gs1, ss0, ss1):
    cid = lax.axis_index("c")
    sid = lax.axis_index("s")
    _prop_rounds([h_hbm], [(o0, o1)], cid, sid, acc_sp, sd3,
                 sdr, (rows0, rows1), (sc0, sc1),
                 (gs0, gs1), (ss0, ss1))


def _prop2_body(ha_hbm, hb_hbm, sd3, oa0, oa1, ob0, ob1,
                acc_sp, sdr, rows0, rows1, sc0, sc1,
                gs0, gs1, ss0, ss1):
    cid = lax.axis_index("c")
    sid = lax.axis_index("s")
    _prop_rounds([ha_hbm, hb_hbm], [(oa0, oa1), (ob0, ob1)],
                 cid, sid, acc_sp, sd3,
                 sdr, (rows0, rows1), (sc0, sc1),
                 (gs0, gs1), (ss0, ss1))


def _prop_scratch():
    f32, i32 = jnp.float32, jnp.int32
    return [
        pltpu.VMEM_SHARED((NP, 128), f32),
        pltpu.VMEM((2, 2, 3, CH64), i32),
        pltpu.VMEM((CH64, 128), f32),
        pltpu.VMEM((CH64, 128), f32),
        pltpu.VMEM((CH64, 128), f32),
        pltpu.VMEM((CH64, 128), f32),
        pltpu.SemaphoreType.DMA,
        pltpu.SemaphoreType.DMA,
        pltpu.SemaphoreType.DMA,
        pltpu.SemaphoreType.DMA,
    ]


def _make_prop1():
    f32 = jnp.float32
    return pl.kernel(
        _prop1_body,
        out_type=(
            jax.ShapeDtypeStruct((NP, 128), f32),
            jax.ShapeDtypeStruct((NP, 128), f32),
        ),
        mesh=plsc.VectorSubcoreMesh(**_MESH),
        scratch_types=_prop_scratch(),
        name="gcn_prop1_sc",
    )


def _make_prop2():
    f32 = jnp.float32
    return pl.kernel(
        _prop2_body,
        out_type=tuple(
            jax.ShapeDtypeStruct((NP, 128), f32) for _ in range(4)),
        mesh=plsc.VectorSubcoreMesh(**_MESH),
        scratch_types=_prop_scratch(),
        name="gcn_prop2_sc",
    )


def _lrelu(v):
    return jnp.where(v >= 0, v, 0.01 * v)


def _t2_body(e1p0, e1p1, x, d2, w1, b1, w2, qa, qb):
    z = e1p0[...] + e1p1[...] + d2[...] * x[...]
    h1 = jnp.dot(z, w1[...], preferred_element_type=jnp.float32) + b1[...]
    h1 = _lrelu(h1)
    q = jnp.dot(h1, w2[...], preferred_element_type=jnp.float32)
    qa[...] = q[:, :128]
    qb[...] = q[:, 128:]


def _make_t2():
    f32 = jnp.float32
    R = 256
    grid = (NP // R,)
    return pl.pallas_call(
        _t2_body,
        grid=grid,
        in_specs=[
            pl.BlockSpec((R, 128), lambda t: (t, 0)),
            pl.BlockSpec((R, 128), lambda t: (t, 0)),
            pl.BlockSpec((R, 128), lambda t: (t, 0)),
            pl.BlockSpec((R, 1), lambda t: (t, 0)),
            pl.BlockSpec((128, 256), lambda t: (0, 0)),
            pl.BlockSpec((1, 256), lambda t: (0, 0)),
            pl.BlockSpec((256, 256), lambda t: (0, 0)),
        ],
        out_specs=[
            pl.BlockSpec((R, 128), lambda t: (t, 0)),
            pl.BlockSpec((R, 128), lambda t: (t, 0)),
        ],
        out_shape=[
            jax.ShapeDtypeStruct((NP, 128), f32),
            jax.ShapeDtypeStruct((NP, 128), f32),
        ],
    )


def _t3_body(e2a0, e2a1, e2b0, e2b1, qa, qb, d2c, b2, bat, d2r, p0, p1,
             w3, b3, fw1, fb1, fw2, fb2, fw3, fb3,
             out, psum, cnt):
    t = pl.program_id(0)
    nt = pl.num_programs(0)

    @pl.when(t == 0)
    def _():
        psum[...] = jnp.zeros_like(psum)
        cnt[...] = jnp.zeros_like(cnt)

    d2v = d2c[...]
    z = jnp.concatenate([e2a0[...] + e2a1[...] + d2v * qa[...],
                         e2b0[...] + e2b1[...] + d2v * qb[...]],
                        axis=1) + b2[...]
    h2 = _lrelu(z)
    g = lax.broadcasted_iota(jnp.int32, (G, 256), 0)
    cmp = bat[...] == g
    mt = p0[...] + p1[...] + jnp.where(cmp, d2r[...], 0.0)
    psum[...] += jnp.dot(mt, h2, preferred_element_type=jnp.float32)
    cnt[:, 0:1] += jnp.sum(cmp.astype(jnp.float32), axis=1, keepdims=True)

    @pl.when(t == nt - 1)
    def _():
        cg = cnt[:, 0:1]
        pooled = psum[...] / jnp.maximum(cg, 1.0)
        h3 = jnp.dot(pooled, w3[...], preferred_element_type=jnp.float32)
        h3 = h3 + jnp.where(cg > 0, b3[...], 0.0)
        z1 = _lrelu(jnp.dot(h3, fw1[...],
                            preferred_element_type=jnp.float32) + fb1[...])
        z2 = _lrelu(jnp.dot(z1, fw2[...],
                            preferred_element_type=jnp.float32) + fb2[...])
        out[...] = jnp.dot(z2, fw3[...],
                           preferred_element_type=jnp.float32) + fb3[...]


def _make_t3():
    f32 = jnp.float32
    R = 256
    grid = (NP // R,)
    return pl.pallas_call(
        _t3_body,
        grid=grid,
        in_specs=[
            pl.BlockSpec((R, 128), lambda t: (t, 0)),
            pl.BlockSpec((R, 128), lambda t: (t, 0)),
            pl.BlockSpec((R, 128), lambda t: (t, 0)),
            pl.BlockSpec((R, 128), lambda t: (t, 0)),
            pl.BlockSpec((R, 128), lambda t: (t, 0)),
            pl.BlockSpec((R, 128), lambda t: (t, 0)),
            pl.BlockSpec((R, 1), lambda t: (t, 0)),
            pl.BlockSpec((1, 256), lambda t: (0, 0)),
            pl.BlockSpec((1, R), lambda t: (0, t)),
            pl.BlockSpec((1, R), lambda t: (0, t)),
            pl.BlockSpec((G, R), lambda t: (0, t)),
            pl.BlockSpec((G, R), lambda t: (0, t)),
            pl.BlockSpec((256, 256), lambda t: (0, 0)),
            pl.BlockSpec((1, 256), lambda t: (0, 0)),
            pl.BlockSpec((256, 128), lambda t: (0, 0)),
            pl.BlockSpec((1, 128), lambda t: (0, 0)),
            pl.BlockSpec((128, 64), lambda t: (0, 0)),
            pl.BlockSpec((1, 64), lambda t: (0, 0)),
            pl.BlockSpec((64, C), lambda t: (0, 0)),
            pl.BlockSpec((1, C), lambda t: (0, 0)),
        ],
        out_specs=pl.BlockSpec((G, C), lambda t: (0, 0)),
        out_shape=jax.ShapeDtypeStruct((G, C), f32),
        scratch_shapes=[
            pltpu.VMEM((G, 256), f32),
            pltpu.VMEM((G, 128), f32),
        ],
    )


_prep = _make_prep()
_prop1 = _make_prop1()
_prop2 = _make_prop2()
_t2 = _make_t2()
_t3 = _make_t3()


def kernel(x, edge_index, edge_weight, batch,
           W1, b1, W2, b2, W3, b3, FW1, Fb1, FW2, Fb2, FW3, Fb3):
    f32 = jnp.float32
    src = edge_index[0]
    dst = edge_index[1]
    x_pad = jnp.pad(x, ((0, NP - N), (0, 0)))
    batch_pad = jnp.pad(batch, (0, NP - N), constant_values=-1)

    d2, c, p0, p1 = _prep(src, dst, edge_weight, batch_pad)

    pad_idx = (N + jnp.arange(E_PAD - E, dtype=jnp.int32) % (NP - N))
    src2d = jnp.concatenate([src, pad_idx]).reshape(ERR, CH64)
    dst2d = jnp.concatenate([dst, pad_idx]).reshape(ERR, CH64)
    c2d = jnp.pad(c, (0, E_PAD - E)).reshape(ERR, CH64)
    sd3 = jnp.stack(
        [src2d, dst2d, lax.bitcast_convert_type(c2d, jnp.int32)], axis=1)

    e1p0, e1p1 = _prop1(x_pad, sd3)

    d2c = d2.reshape(NP, 1)
    qa, qb = _t2(e1p0, e1p1, x_pad, d2c, W1, b1.reshape(1, H), W2)

    e2a0, e2a1, e2b0, e2b1 = _prop2(qa, qb, sd3)

    out = _t3(e2a0, e2a1, e2b0, e2b1, qa, qb, d2c, b2.reshape(1, H),
              batch_pad.reshape(1, NP), d2.reshape(1, NP),
              p0.reshape(G, NP), p1.reshape(G, NP),
              W3, b3.reshape(1, H), FW1, Fb1.reshape(1, H // 2),
              FW2, Fb2.reshape(1, H // 4), FW3, Fb3.reshape(1, C))
    return out


# speedup vs baseline: 1.9917x; 1.3476x over previous
import functools

import jax
import jax.numpy as jnp
from jax import lax
from jax.experimental import pallas as pl
from jax.experimental.pallas import tpu as pltpu
from jax.experimental.pallas import tpu_sc as plsc

N = 10000
E = 320000
D_IN = 128
H = 256
C = 40
G = 64
NP = 10240

NSC = 2
NSUB = 16

CH = 128

EPT16 = E // NSUB
EPT32 = E // (NSC * NSUB)

_MESH = dict(core_axis_name="c", subcore_axis_name="s")


def _zero_vec():
    return jnp.zeros((16,), jnp.float32)


import numpy as _np
_BCAST = [_np.full((16,), i, _np.int32) for i in range(16)]


def _lane_bcast(cv, e16):
    idx = lax.iota(jnp.int32, 16) * 0 + e16
    return lax.gather(
        cv, idx[:, None],
        dimension_numbers=lax.GatherDimensionNumbers(
            offset_dims=(), collapsed_slice_dims=(0,), start_index_map=(0,)),
        slice_sizes=(1,), mode=lax.GatherScatterMode.PROMISE_IN_BOUNDS)

def _fisr(d):
    i = lax.bitcast_convert_type(d, jnp.int32)
    y = lax.bitcast_convert_type(
        jnp.int32(0x5F3759DF) - lax.shift_right_logical(i, 1), jnp.float32)
    for _ in range(4):
        y = y * (1.5 - 0.5 * d * y * y)
    return y


def _prep_body(dst128, ew128, src128, batch_hbm,
               d2_hbm, c2d_hbm, p0_hbm, p1_hbm,
               deg_sp, dis_sp, bat_sp, p_sp,
               zvm, dst_all, ew_all, s_all, dsb, ddb, bbv, pvm,
               cvm_all, disbuf, d2buf, dsem, gs0, gs1, ps0, ps1):
    cid = lax.axis_index("c")
    sid = lax.axis_index("s")
    wid = sid * NSC + cid
    DR = 160
    CR = 80
    cb0 = cid * CR

    def z_body(i, _):
        zvm[pl.ds(i * 16, 16)] = _zero_vec()
        return 0
    lax.fori_loop(0, 160, z_body, 0)
    pltpu.sync_copy(zvm.at[pl.ds(0, 640)], deg_sp.at[pl.ds(sid * 640, 640)])
    for k in range(16):
        pltpu.sync_copy(zvm, p_sp.at[pl.ds(sid * 40960 + k * 2560, 2560)])

    @pl.when(sid == 0)
    def _():
        pltpu.sync_copy(batch_hbm, bat_sp)

    pltpu.sync_copy(dst128.at[pl.ds(sid * DR, DR)], dst_all)
    pltpu.sync_copy(ew128.at[pl.ds(sid * DR, DR)], ew_all)
    plsc.subcore_barrier()

    def deg_group(g, _):
        descs = []
        for u in range(8):
            j = g * 8 + u
            descs.append(pltpu.async_copy(
                ew_all.at[j], deg_sp.at[dst_all.at[j]], dsem, add=True))
        for d in descs:
            d.wait()
        return 0
    lax.fori_loop(0, DR // 8, deg_group, 0)
    plsc.subcore_barrier()

    pltpu.sync_copy(deg_sp.at[pl.ds(sid * 640, 640)], disbuf)

    def dis_body(i, _):
        d = disbuf[pl.ds(i * 16, 16)] + 1.0
        y = _fisr(d)
        disbuf[pl.ds(i * 16, 16)] = y
        d2buf[pl.ds(i * 16, 16)] = y * y
        return 0
    lax.fori_loop(0, 40, dis_body, 0)
    pltpu.sync_copy(disbuf, dis_sp.at[pl.ds(sid * 640, 640)])

    @pl.when(cid == 0)
    def _():
        pltpu.sync_copy(d2buf, d2_hbm.at[pl.ds(sid * 640, 640)])
    plsc.subcore_barrier()

    pltpu.sync_copy(src128.at[pl.ds(wid * CR, CR)], s_all)
    gsem = (gs0, gs1)
    psem = (ps0, ps1)

    def fire3(j, b):
        pltpu.async_copy(dis_sp.at[s_all.at[j]], dsb.at[b], gsem[b])
        pltpu.async_copy(dis_sp.at[dst_all.at[cb0 + j]], ddb.at[b], gsem[b])
        pltpu.async_copy(bat_sp.at[dst_all.at[cb0 + j]], bbv.at[b], gsem[b])

    def wait3(j, b):
        pltpu.make_async_copy(dis_sp.at[s_all.at[j]], dsb.at[b],
                              gsem[b]).wait()
        pltpu.make_async_copy(dis_sp.at[dst_all.at[cb0 + j]], ddb.at[b],
                              gsem[b]).wait()
        pltpu.make_async_copy(bat_sp.at[dst_all.at[cb0 + j]], bbv.at[b],
                              gsem[b]).wait()

    def fire_p(j, b):
        pltpu.async_copy(cvm_all.at[j], p_sp.at[pvm.at[b]], psem[b],
                         add=True)

    def wait_p(j, b):
        pltpu.make_async_copy(cvm_all.at[j], p_sp.at[pvm.at[b]],
                              psem[b]).wait()

    fire3(0, 0)
    fire3(1, 1)

    def c_pair(jp, _):
        for b in range(2):
            j = 2 * jp + b
            wait3(j, b)
            for k in range(8):
                sl = pl.ds(k * 16, 16)
                c16 = ew_all[cb0 + j, sl] * dsb[b, sl] * ddb[b, sl]
                cvm_all[j, sl] = c16
                pvm[b, sl] = bbv[b, sl] * NP + s_all[j, sl]

            @pl.when(j + 2 < CR)
            def _():
                fire3(j + 2, b)

            @pl.when(jp >= 1)
            def _():
                wait_p(j - 2, b)
            fire_p(j, b)
        return 0
    lax.fori_loop(0, CR // 2, c_pair, 0)
    wait_p(CR - 2, 0)
    wait_p(CR - 1, 1)
    pltpu.sync_copy(cvm_all, c2d_hbm.at[pl.ds(wid * CR, CR)])
    plsc.subcore_barrier()

    @pl.when(cid == 0)
    def _():
        pltpu.sync_copy(p_sp.at[pl.ds(sid * 40960, 40960)],
                        p0_hbm.at[pl.ds(sid * 40960, 40960)])

    @pl.when(cid == 1)
    def _():
        pltpu.sync_copy(p_sp.at[pl.ds(sid * 40960, 40960)],
                        p1_hbm.at[pl.ds(sid * 40960, 40960)])


def _make_prep():
    f32, i32 = jnp.float32, jnp.int32
    return pl.kernel(
        _prep_body,
        out_type=(
            jax.ShapeDtypeStruct((NP,), f32),
            jax.ShapeDtypeStruct((2560, 128), f32),
            jax.ShapeDtypeStruct((G * NP,), f32),
            jax.ShapeDtypeStruct((G * NP,), f32),
        ),
        mesh=plsc.VectorSubcoreMesh(**_MESH),
        scratch_types=[
            pltpu.VMEM_SHARED((NP,), f32),
            pltpu.VMEM_SHARED((NP,), f32),
            pltpu.VMEM_SHARED((NP,), i32),
            pltpu.VMEM_SHARED((G * NP,), f32),
            pltpu.VMEM((2560,), f32),
            pltpu.VMEM((160, 128), i32),
            pltpu.VMEM((160, 128), f32),
            pltpu.VMEM((80, 128), i32),
            pltpu.VMEM((2, 128), f32),
            pltpu.VMEM((2, 128), f32),
            pltpu.VMEM((2, 128), i32),
            pltpu.VMEM((2, 128), i32),
            pltpu.VMEM((80, 128), f32),
            pltpu.VMEM((640,), f32),
            pltpu.VMEM((640,), f32),
            pltpu.SemaphoreType.DMA,
            pltpu.SemaphoreType.DMA,
            pltpu.SemaphoreType.DMA,
            pltpu.SemaphoreType.DMA,
            pltpu.SemaphoreType.DMA,
        ],
        name="gcn_prep_sc",
    )


CH64 = 64
ERR = 5120
CPT = ERR // 32
E_PAD = ERR * CH64


def _prop_rounds(h_list, out_list, cid, sid, acc_sp,
                 sd3, sdr, rows, scaled, gsem, ssem):
    wid = sid * NSC + cid
    gbase = wid * CPT

    def fire_gather(h_hbm, ps, b):
        return pltpu.async_copy(h_hbm.at[sdr.at[ps, b, 0]], rows[b], gsem[b])

    def wait_gather(h_hbm, ps, b):
        pltpu.make_async_copy(h_hbm.at[sdr.at[ps, b, 0]], rows[b],
                              gsem[b]).wait()

    def fire_scatter(ps, b):
        return pltpu.async_copy(scaled[b], acc_sp.at[sdr.at[ps, b, 1]],
                                ssem[b], add=True)

    def wait_scatter(ps, b):
        pltpu.make_async_copy(scaled[b], acc_sp.at[sdr.at[ps, b, 1]],
                              ssem[b]).wait()

    def copy_sd(pr, ps):
        pltpu.sync_copy(sd3.at[pl.ds(gbase + 2 * pr, 2)], sdr.at[ps])

    def scale(ps, b):
        def sj(j, _):
            cv = lax.bitcast_convert_type(
                sdr[ps, b, 2, pl.ds(j * 16, 16)], jnp.float32)
            for e16 in range(16):
                e = j * 16 + e16
                cb = _lane_bcast(cv, e16)
                for k in range(8):
                    scaled[b][e, pl.ds(k * 16, 16)] = (
                        rows[b][e, pl.ds(k * 16, 16)] * cb)
            return 0
        lax.fori_loop(0, CH64 // 16, sj, 0)

    for r, (h_hbm, (o0_hbm, o1_hbm)) in enumerate(zip(h_list, out_list)):
        def z_body(i, _):
            scaled[0][i // 8, pl.ds((i % 8) * 16, 16)] = _zero_vec()
            return 0
        lax.fori_loop(0, CH64 * 8, z_body, 0)
        for k in range(10):
            pltpu.sync_copy(scaled[0],
                            acc_sp.at[pl.ds(sid * 640 + k * 64, 64)])
        plsc.subcore_barrier()

        copy_sd(0, 0)
        copy_sd(1, 1)
        fire_gather(h_hbm, 0, 0)
        fire_gather(h_hbm, 0, 1)

        def q_body(q, _):
            for bq in range(2):
                pr = 2 * q + bq
                ps = bq
                for b in range(2):
                    wait_gather(h_hbm, ps, b)

                    @pl.when(pr >= 1)
                    def _():
                        wait_scatter(ps, b)
                    scale(ps, b)
                    fire_scatter(ps, b)

                    @pl.when(pr + 1 < CPT // 2)
                    def _():
                        fire_gather(h_hbm, 1 - ps, b)

                @pl.when(pr + 2 < CPT // 2)
                def _():
                    copy_sd(pr + 2, ps)
            return 0
        lax.fori_loop(0, CPT // 4, q_body, 0)
        wait_scatter(1, 0)
        wait_scatter(1, 1)
        plsc.subcore_barrier()

        @pl.when(cid == 0)
        def _():
            pltpu.sync_copy(acc_sp.at[pl.ds(sid * 640, 640)],
                            o0_hbm.at[pl.ds(sid * 640, 640)])

        @pl.when(cid == 1)
        def _():
            pltpu.sync_copy(acc_sp.at[pl.ds(sid * 640, 640)],
                            o1_hbm.at[pl.ds(sid * 640, 640)])
        if r + 1 < len(h_list):
            plsc.subcore_barrier()


def _prop1_body(h_hbm, sd3, o0, o1,
                acc_sp, sdr, rows0, rows1, sc0, sc1,
                gs0, gs1, ss0, ss1):
    cid = lax.axis_index("c")
    sid = lax.axis_index("s")
    _prop_rounds([h_hbm], [(o0, o1)], cid, sid, acc_sp, sd3,
                 sdr, (rows0, rows1), (sc0, sc1),
                 (gs0, gs1), (ss0, ss1))


def _prop2_body(ha_hbm, hb_hbm, sd3, oa0, oa1, ob0, ob1,
                acc_sp, sdr, rows0, rows1, sc0, sc1,
                gs0, gs1, ss0, ss1):
    cid = lax.axis_index("c")
    sid = lax.axis_index("s")
    _prop_rounds([ha_hbm, hb_hbm], [(oa0, oa1), (ob0, ob1)],
                 cid, sid, acc_sp, sd3,
                 sdr, (rows0, rows1), (sc0, sc1),
                 (gs0, gs1), (ss0, ss1))


def _prop_scratch():
    f32, i32 = jnp.float32, jnp.int32
    return [
        pltpu.VMEM_SHARED((NP, 128), f32),
        pltpu.VMEM((2, 2, 3, CH64), i32),
        pltpu.VMEM((CH64, 128), f32),
        pltpu.VMEM((CH64, 128), f32),
        pltpu.VMEM((CH64, 128), f32),
        pltpu.VMEM((CH64, 128), f32),
        pltpu.SemaphoreType.DMA,
        pltpu.SemaphoreType.DMA,
        pltpu.SemaphoreType.DMA,
        pltpu.SemaphoreType.DMA,
    ]


def _make_prop1():
    f32 = jnp.float32
    return pl.kernel(
        _prop1_body,
        out_type=(
            jax.ShapeDtypeStruct((NP, 128), f32),
            jax.ShapeDtypeStruct((NP, 128), f32),
        ),
        mesh=plsc.VectorSubcoreMesh(**_MESH),
        scratch_types=_prop_scratch(),
        name="gcn_prop1_sc",
    )


def _make_prop2():
    f32 = jnp.float32
    return pl.kernel(
        _prop2_body,
        out_type=tuple(
            jax.ShapeDtypeStruct((NP, 128), f32) for _ in range(4)),
        mesh=plsc.VectorSubcoreMesh(**_MESH),
        scratch_types=_prop_scratch(),
        name="gcn_prop2_sc",
    )


def _lrelu(v):
    return jnp.where(v >= 0, v, 0.01 * v)


def _t2_body(e1p0, e1p1, x, d2, w1, b1, w2, qa, qb):
    z = e1p0[...] + e1p1[...] + d2[...] * x[...]
    h1 = jnp.dot(z, w1[...], preferred_element_type=jnp.float32) + b1[...]
    h1 = _lrelu(h1)
    q = jnp.dot(h1, w2[...], preferred_element_type=jnp.float32)
    qa[...] = q[:, :128]
    qb[...] = q[:, 128:]


def _make_t2():
    f32 = jnp.float32
    R = 256
    grid = (NP // R,)
    return pl.pallas_call(
        _t2_body,
        grid=grid,
        in_specs=[
            pl.BlockSpec((R, 128), lambda t: (t, 0)),
            pl.BlockSpec((R, 128), lambda t: (t, 0)),
            pl.BlockSpec((R, 128), lambda t: (t, 0)),
            pl.BlockSpec((R, 1), lambda t: (t, 0)),
            pl.BlockSpec((128, 256), lambda t: (0, 0)),
            pl.BlockSpec((1, 256), lambda t: (0, 0)),
            pl.BlockSpec((256, 256), lambda t: (0, 0)),
        ],
        out_specs=[
            pl.BlockSpec((R, 128), lambda t: (t, 0)),
            pl.BlockSpec((R, 128), lambda t: (t, 0)),
        ],
        out_shape=[
            jax.ShapeDtypeStruct((NP, 128), f32),
            jax.ShapeDtypeStruct((NP, 128), f32),
        ],
    )


def _t3_body(e2a0, e2a1, e2b0, e2b1, qa, qb, d2c, b2, bat, d2r, p0, p1,
             w3, b3, fw1, fb1, fw2, fb2, fw3, fb3,
             out, psum, cnt):
    t = pl.program_id(0)
    nt = pl.num_programs(0)

    @pl.when(t == 0)
    def _():
        psum[...] = jnp.zeros_like(psum)
        cnt[...] = jnp.zeros_like(cnt)

    d2v = d2c[...]
    z = jnp.concatenate([e2a0[...] + e2a1[...] + d2v * qa[...],
                         e2b0[...] + e2b1[...] + d2v * qb[...]],
                        axis=1) + b2[...]
    h2 = _lrelu(z)
    g = lax.broadcasted_iota(jnp.int32, (G, 256), 0)
    cmp = bat[...] == g
    mt = p0[...] + p1[...] + jnp.where(cmp, d2r[...], 0.0)
    psum[...] += jnp.dot(mt, h2, preferred_element_type=jnp.float32)
    cnt[:, 0:1] += jnp.sum(cmp.astype(jnp.float32), axis=1, keepdims=True)

    @pl.when(t == nt - 1)
    def _():
        cg = cnt[:, 0:1]
        pooled = psum[...] / jnp.maximum(cg, 1.0)
        h3 = jnp.dot(pooled, w3[...], preferred_element_type=jnp.float32)
        h3 = h3 + jnp.where(cg > 0, b3[...], 0.0)
        z1 = _lrelu(jnp.dot(h3, fw1[...],
                            preferred_element_type=jnp.float32) + fb1[...])
        z2 = _lrelu(jnp.dot(z1, fw2[...],
                            preferred_element_type=jnp.float32) + fb2[...])
        out[...] = jnp.dot(z2, fw3[...],
                           preferred_element_type=jnp.float32) + fb3[...]


def _make_t3():
    f32 = jnp.float32
    R = 256
    grid = (NP // R,)
    return pl.pallas_call(
        _t3_body,
        grid=grid,
        in_specs=[
            pl.BlockSpec((R, 128), lambda t: (t, 0)),
            pl.BlockSpec((R, 128), lambda t: (t, 0)),
            pl.BlockSpec((R, 128), lambda t: (t, 0)),
            pl.BlockSpec((R, 128), lambda t: (t, 0)),
            pl.BlockSpec((R, 128), lambda t: (t, 0)),
            pl.BlockSpec((R, 128), lambda t: (t, 0)),
            pl.BlockSpec((R, 1), lambda t: (t, 0)),
            pl.BlockSpec((1, 256), lambda t: (0, 0)),
            pl.BlockSpec((1, R), lambda t: (0, t)),
            pl.BlockSpec((1, R), lambda t: (0, t)),
            pl.BlockSpec((G, R), lambda t: (0, t)),
            pl.BlockSpec((G, R), lambda t: (0, t)),
            pl.BlockSpec((256, 256), lambda t: (0, 0)),
            pl.BlockSpec((1, 256), lambda t: (0, 0)),
            pl.BlockSpec((256, 128), lambda t: (0, 0)),
            pl.BlockSpec((1, 128), lambda t: (0, 0)),
            pl.BlockSpec((128, 64), lambda t: (0, 0)),
            pl.BlockSpec((1, 64), lambda t: (0, 0)),
            pl.BlockSpec((64, C), lambda t: (0, 0)),
            pl.BlockSpec((1, C), lambda t: (0, 0)),
        ],
        out_specs=pl.BlockSpec((G, C), lambda t: (0, 0)),
        out_shape=jax.ShapeDtypeStruct((G, C), f32),
        scratch_shapes=[
            pltpu.VMEM((G, 256), f32),
            pltpu.VMEM((G, 128), f32),
        ],
    )


_prep = _make_prep()
_prop1 = _make_prop1()
_prop2 = _make_prop2()
_t2 = _make_t2()
_t3 = _make_t3()


def kernel(x, edge_index, edge_weight, batch,
           W1, b1, W2, b2, W3, b3, FW1, Fb1, FW2, Fb2, FW3, Fb3):
    f32 = jnp.float32
    src = edge_index[0]
    dst = edge_index[1]
    x_pad = jnp.pad(x, ((0, NP - N), (0, 0)))
    batch_pad = jnp.pad(batch, (0, NP - N), constant_values=-1)

    pad_idx = (N + jnp.arange(E_PAD - E, dtype=jnp.int32) % (NP - N))
    src_p = jnp.concatenate([src, pad_idx])
    dst_p = jnp.concatenate([dst, pad_idx])
    ew_p = jnp.pad(edge_weight, (0, E_PAD - E))
    batch_sc = jnp.pad(batch, (0, NP - N))

    dst128 = dst_p.reshape(2560, 128)
    ew128 = ew_p.reshape(2560, 128)
    src128 = src_p.reshape(2560, 128)

    d2, c2d, p0, p1 = _prep(dst128, ew128, src128, batch_sc)

    sd3 = jnp.stack(
        [src_p.reshape(ERR, CH64), dst_p.reshape(ERR, CH64),
         lax.bitcast_convert_type(c2d.reshape(-1), jnp.int32).reshape(
             ERR, CH64)], axis=1)

    e1p0, e1p1 = _prop1(x_pad, sd3)

    d2c = d2.reshape(NP, 1)
    qa, qb = _t2(e1p0, e1p1, x_pad, d2c, W1, b1.reshape(1, H), W2)

    e2a0, e2a1, e2b0, e2b1 = _prop2(qa, qb, sd3)

    out = _t3(e2a0, e2a1, e2b0, e2b1, qa, qb, d2c, b2.reshape(1, H),
              batch_pad.reshape(1, NP), d2.reshape(1, NP),
              p0.reshape(G, NP), p1.reshape(G, NP),
              W3, b3.reshape(1, H), FW1, Fb1.reshape(1, H // 2),
              FW2, Fb2.reshape(1, H // 4), FW3, Fb3.reshape(1, C))
    return out


# speedup vs baseline: 1.9944x; 1.0014x over previous
import functools

import jax
import jax.numpy as jnp
from jax import lax
from jax.experimental import pallas as pl
from jax.experimental.pallas import tpu as pltpu
from jax.experimental.pallas import tpu_sc as plsc

N = 10000
E = 320000
D_IN = 128
H = 256
C = 40
G = 64
NP = 10240

NSC = 2
NSUB = 16

CH = 128

EPT16 = E // NSUB
EPT32 = E // (NSC * NSUB)

_MESH = dict(core_axis_name="c", subcore_axis_name="s")


def _zero_vec():
    return jnp.zeros((16,), jnp.float32)


import numpy as _np

_DINV = _np.empty((128,), _np.int32)
for _k in range(4):
    for _t in range(16):
        _DINV[32 * _k + 2 * _t] = 32 * _k + _t
        _DINV[32 * _k + 2 * _t + 1] = 32 * _k + 16 + _t


def _lane_bcast(cv, e16):
    idx = lax.iota(jnp.int32, 16) * 0 + e16
    return lax.gather(
        cv, idx[:, None],
        dimension_numbers=lax.GatherDimensionNumbers(
            offset_dims=(), collapsed_slice_dims=(0,), start_index_map=(0,)),
        slice_sizes=(1,), mode=lax.GatherScatterMode.PROMISE_IN_BOUNDS)

def _fisr(d):
    i = lax.bitcast_convert_type(d, jnp.int32)
    y = lax.bitcast_convert_type(
        jnp.int32(0x5F3759DF) - lax.shift_right_logical(i, 1), jnp.float32)
    for _ in range(4):
        y = y * (1.5 - 0.5 * d * y * y)
    return y


def _prep_body(dst128, ew128, src128, batch_hbm,
               d2_hbm, c2d_hbm, p0_hbm, p1_hbm,
               deg_sp, dis_sp, bat_sp, p_sp,
               zvm, dst_all, ew_all, s_all, dsb, ddb, bbv, pvm,
               cvm_all, disbuf, d2buf, dsem, gs0, gs1, ps0, ps1):
    cid = lax.axis_index("c")
    sid = lax.axis_index("s")
    wid = sid * NSC + cid
    DR = 160
    CR = 80
    cb0 = cid * CR

    def z_body(i, _):
        zvm[pl.ds(i * 16, 16)] = _zero_vec()
        return 0
    lax.fori_loop(0, 160, z_body, 0)
    pltpu.sync_copy(zvm.at[pl.ds(0, 640)], deg_sp.at[pl.ds(sid * 640, 640)])
    for k in range(16):
        pltpu.sync_copy(zvm, p_sp.at[pl.ds(sid * 40960 + k * 2560, 2560)])

    @pl.when(sid == 0)
    def _():
        pltpu.sync_copy(batch_hbm, bat_sp)

    pltpu.sync_copy(dst128.at[pl.ds(sid * DR, DR)], dst_all)
    pltpu.sync_copy(ew128.at[pl.ds(sid * DR, DR)], ew_all)
    plsc.subcore_barrier()

    def deg_group(g, _):
        descs = []
        for u in range(8):
            j = g * 8 + u
            descs.append(pltpu.async_copy(
                ew_all.at[j], deg_sp.at[dst_all.at[j]], dsem, add=True))
        for d in descs:
            d.wait()
        return 0
    lax.fori_loop(0, DR // 8, deg_group, 0)
    plsc.subcore_barrier()

    pltpu.sync_copy(deg_sp.at[pl.ds(sid * 640, 640)], disbuf)

    def dis_body(i, _):
        d = disbuf[pl.ds(i * 16, 16)] + 1.0
        y = _fisr(d)
        disbuf[pl.ds(i * 16, 16)] = y
        d2buf[pl.ds(i * 16, 16)] = y * y
        return 0
    lax.fori_loop(0, 40, dis_body, 0)
    pltpu.sync_copy(disbuf, dis_sp.at[pl.ds(sid * 640, 640)])

    @pl.when(cid == 0)
    def _():
        pltpu.sync_copy(d2buf, d2_hbm.at[pl.ds(sid * 640, 640)])
    plsc.subcore_barrier()

    pltpu.sync_copy(src128.at[pl.ds(wid * CR, CR)], s_all)
    gsem = (gs0, gs1)
    psem = (ps0, ps1)

    def fire3(j, b):
        pltpu.async_copy(dis_sp.at[s_all.at[j]], dsb.at[b], gsem[b])
        pltpu.async_copy(dis_sp.at[dst_all.at[cb0 + j]], ddb.at[b], gsem[b])
        pltpu.async_copy(bat_sp.at[dst_all.at[cb0 + j]], bbv.at[b], gsem[b])

    def wait3(j, b):
        pltpu.make_async_copy(dis_sp.at[s_all.at[j]], dsb.at[b],
                              gsem[b]).wait()
        pltpu.make_async_copy(dis_sp.at[dst_all.at[cb0 + j]], ddb.at[b],
                              gsem[b]).wait()
        pltpu.make_async_copy(bat_sp.at[dst_all.at[cb0 + j]], bbv.at[b],
                              gsem[b]).wait()

    def fire_p(j, b):
        pltpu.async_copy(cvm_all.at[j], p_sp.at[pvm.at[b]], psem[b],
                         add=True)

    def wait_p(j, b):
        pltpu.make_async_copy(cvm_all.at[j], p_sp.at[pvm.at[b]],
                              psem[b]).wait()

    fire3(0, 0)
    fire3(1, 1)

    def c_pair(jp, _):
        for b in range(2):
            j = 2 * jp + b
            wait3(j, b)
            for k in range(8):
                sl = pl.ds(k * 16, 16)
                c16 = ew_all[cb0 + j, sl] * dsb[b, sl] * ddb[b, sl]
                cvm_all[j, sl] = c16
                pvm[b, sl] = bbv[b, sl] * NP + s_all[j, sl]

            @pl.when(j + 2 < CR)
            def _():
                fire3(j + 2, b)

            @pl.when(jp >= 1)
            def _():
                wait_p(j - 2, b)
            fire_p(j, b)
        return 0
    lax.fori_loop(0, CR // 2, c_pair, 0)
    wait_p(CR - 2, 0)
    wait_p(CR - 1, 1)
    pltpu.sync_copy(cvm_all, c2d_hbm.at[pl.ds(wid * CR, CR)])
    plsc.subcore_barrier()

    @pl.when(cid == 0)
    def _():
        pltpu.sync_copy(p_sp.at[pl.ds(sid * 40960, 40960)],
                        p0_hbm.at[pl.ds(sid * 40960, 40960)])

    @pl.when(cid == 1)
    def _():
        pltpu.sync_copy(p_sp.at[pl.ds(sid * 40960, 40960)],
                        p1_hbm.at[pl.ds(sid * 40960, 40960)])


def _make_prep():
    f32, i32 = jnp.float32, jnp.int32
    return pl.kernel(
        _prep_body,
        out_type=(
            jax.ShapeDtypeStruct((NP,), f32),
            jax.ShapeDtypeStruct((2560, 128), f32),
            jax.ShapeDtypeStruct((G * NP,), f32),
            jax.ShapeDtypeStruct((G * NP,), f32),
        ),
        mesh=plsc.VectorSubcoreMesh(**_MESH),
        scratch_types=[
            pltpu.VMEM_SHARED((NP,), f32),
            pltpu.VMEM_SHARED((NP,), f32),
            pltpu.VMEM_SHARED((NP,), i32),
            pltpu.VMEM_SHARED((G * NP,), f32),
            pltpu.VMEM((2560,), f32),
            pltpu.VMEM((160, 128), i32),
            pltpu.VMEM((160, 128), f32),
            pltpu.VMEM((80, 128), i32),
            pltpu.VMEM((2, 128), f32),
            pltpu.VMEM((2, 128), f32),
            pltpu.VMEM((2, 128), i32),
            pltpu.VMEM((2, 128), i32),
            pltpu.VMEM((80, 128), f32),
            pltpu.VMEM((640,), f32),
            pltpu.VMEM((640,), f32),
            pltpu.SemaphoreType.DMA,
            pltpu.SemaphoreType.DMA,
            pltpu.SemaphoreType.DMA,
            pltpu.SemaphoreType.DMA,
            pltpu.SemaphoreType.DMA,
        ],
        name="gcn_prep_sc",
    )


CH64 = 64
ERR = 5120
CPT = ERR // 32
E_PAD = ERR * CH64


def _prop_rounds(h_list, out_list, cid, sid, acc_sp,
                 sd3, sdr, rows, scaled, gsem, ssem):
    wid = sid * NSC + cid
    gbase = wid * CPT

    def fire_gather(h_hbm, ps, b):
        return pltpu.async_copy(h_hbm.at[sdr.at[ps, b, 0]], rows[b], gsem[b])

    def wait_gather(h_hbm, ps, b):
        pltpu.make_async_copy(h_hbm.at[sdr.at[ps, b, 0]], rows[b],
                              gsem[b]).wait()

    def fire_scatter(ps, b):
        return pltpu.async_copy(scaled[b], acc_sp.at[sdr.at[ps, b, 1]],
                                ssem[b], add=True)

    def wait_scatter(ps, b):
        pltpu.make_async_copy(scaled[b], acc_sp.at[sdr.at[ps, b, 1]],
                              ssem[b]).wait()

    def copy_sd(pr, ps):
        pltpu.sync_copy(sd3.at[pl.ds(gbase + 2 * pr, 2)], sdr.at[ps])

    def scale(ps, b):
        def sj(j, _):
            cv = lax.bitcast_convert_type(
                sdr[ps, b, 2, pl.ds(j * 16, 16)], jnp.float32)
            for e16 in range(16):
                e = j * 16 + e16
                cb = _lane_bcast(cv, e16)
                for k in range(8):
                    scaled[b][e, pl.ds(k * 16, 16)] = (
                        rows[b][e, pl.ds(k * 16, 16)] * cb)
            return 0
        lax.fori_loop(0, CH64 // 16, sj, 0)

    for r, (h_hbm, (o0_hbm, o1_hbm)) in enumerate(zip(h_list, out_list)):
        def z_body(i, _):
            scaled[0][i // 8, pl.ds((i % 8) * 16, 16)] = _zero_vec()
            return 0
        lax.fori_loop(0, CH64 * 8, z_body, 0)
        for k in range(10):
            pltpu.sync_copy(scaled[0],
                            acc_sp.at[pl.ds(sid * 640 + k * 64, 64)])
        plsc.subcore_barrier()

        copy_sd(0, 0)
        copy_sd(1, 1)
        fire_gather(h_hbm, 0, 0)
        fire_gather(h_hbm, 0, 1)

        def q_body(q, _):
            for bq in range(2):
                pr = 2 * q + bq
                ps = bq
                for b in range(2):
                    wait_gather(h_hbm, ps, b)

                    @pl.when(pr >= 1)
                    def _():
                        wait_scatter(ps, b)
                    scale(ps, b)
                    fire_scatter(ps, b)

                    @pl.when(pr + 1 < CPT // 2)
                    def _():
                        fire_gather(h_hbm, 1 - ps, b)

                @pl.when(pr + 2 < CPT // 2)
                def _():
                    copy_sd(pr + 2, ps)
            return 0
        lax.fori_loop(0, CPT // 4, q_body, 0)
        wait_scatter(1, 0)
        wait_scatter(1, 1)
        plsc.subcore_barrier()

        @pl.when(cid == 0)
        def _():
            pltpu.sync_copy(acc_sp.at[pl.ds(sid * 640, 640)],
                            o0_hbm.at[pl.ds(sid * 640, 640)])

        @pl.when(cid == 1)
        def _():
            pltpu.sync_copy(acc_sp.at[pl.ds(sid * 640, 640)],
                            o1_hbm.at[pl.ds(sid * 640, 640)])
        if r + 1 < len(h_list):
            plsc.subcore_barrier()


def _prop1_body(h_hbm, sd3, o0, o1,
                acc_sp, sdr, rows0, rows1, sc0, sc1,
                gs0, gs1, ss0, ss1):
    cid = lax.axis_index("c")
    sid = lax.axis_index("s")
    _prop_rounds([h_hbm], [(o0, o1)], cid, sid, acc_sp, sd3,
                 sdr, (rows0, rows1), (sc0, sc1),
                 (gs0, gs1), (ss0, ss1))


def _prop2_body(ha_hbm, hb_hbm, sd3, oa0, oa1, ob0, ob1,
                acc_sp, sdr, rows0, rows1, sc0, sc1,
                gs0, gs1, ss0, ss1):
    cid = lax.axis_index("c")
    sid = lax.axis_index("s")
    _prop_rounds([ha_hbm, hb_hbm], [(oa0, oa1), (ob0, ob1)],
                 cid, sid, acc_sp, sd3,
                 sdr, (rows0, rows1), (sc0, sc1),
                 (gs0, gs1), (ss0, ss1))


def _prop_scratch():
    f32, i32 = jnp.float32, jnp.int32
    return [
        pltpu.VMEM_SHARED((NP, 128), f32),
        pltpu.VMEM((2, 2, 3, CH64), i32),
        pltpu.VMEM((CH64, 128), f32),
        pltpu.VMEM((CH64, 128), f32),
        pltpu.VMEM((CH64, 128), f32),
        pltpu.VMEM((CH64, 128), f32),
        pltpu.SemaphoreType.DMA,
        pltpu.SemaphoreType.DMA,
        pltpu.SemaphoreType.DMA,
        pltpu.SemaphoreType.DMA,
    ]


def _make_prop1():
    f32 = jnp.float32
    return pl.kernel(
        _prop1_body,
        out_type=(
            jax.ShapeDtypeStruct((NP, 128), f32),
            jax.ShapeDtypeStruct((NP, 128), f32),
        ),
        mesh=plsc.VectorSubcoreMesh(**_MESH),
        scratch_types=_prop_scratch(),
        name="gcn_prop1_sc",
    )


def _make_prop2():
    f32 = jnp.float32
    return pl.kernel(
        _prop2_body,
        out_type=tuple(
            jax.ShapeDtypeStruct((NP, 128), f32) for _ in range(4)),
        mesh=plsc.VectorSubcoreMesh(**_MESH),
        scratch_types=_prop_scratch(),
        name="gcn_prop2_sc",
    )


def _lrelu(v):
    return jnp.where(v >= 0, v, 0.01 * v)


def _t2_body(e1p0, e1p1, x, d2, w1, b1, w2, qa, qb):
    z = e1p0[...] + e1p1[...] + d2[...] * x[...]
    h1 = jnp.dot(z, w1[...], preferred_element_type=jnp.float32) + b1[...]
    h1 = _lrelu(h1)
    q = jnp.dot(h1, w2[...], preferred_element_type=jnp.float32)
    qa[...] = q[:, :128]
    qb[...] = q[:, 128:]


def _make_t2():
    f32 = jnp.float32
    R = 256
    grid = (NP // R,)
    return pl.pallas_call(
        _t2_body,
        grid=grid,
        in_specs=[
            pl.BlockSpec((R, 128), lambda t: (t, 0)),
            pl.BlockSpec((R, 128), lambda t: (t, 0)),
            pl.BlockSpec((R, 128), lambda t: (t, 0)),
            pl.BlockSpec((R, 1), lambda t: (t, 0)),
            pl.BlockSpec((128, 256), lambda t: (0, 0)),
            pl.BlockSpec((1, 256), lambda t: (0, 0)),
            pl.BlockSpec((256, 256), lambda t: (0, 0)),
        ],
        out_specs=[
            pl.BlockSpec((R, 128), lambda t: (t, 0)),
            pl.BlockSpec((R, 128), lambda t: (t, 0)),
        ],
        out_shape=[
            jax.ShapeDtypeStruct((NP, 128), f32),
            jax.ShapeDtypeStruct((NP, 128), f32),
        ],
    )


def _t3_body(e2a0, e2a1, e2b0, e2b1, qa, qb, d2c, b2, bat, d2r, p0, p1,
             w3, b3, fw1, fb1, fw2, fb2, fw3, fb3,
             out, psum, cnt):
    t = pl.program_id(0)
    nt = pl.num_programs(0)

    @pl.when(t == 0)
    def _():
        psum[...] = jnp.zeros_like(psum)
        cnt[...] = jnp.zeros_like(cnt)

    d2v = d2c[...]
    z = jnp.concatenate([e2a0[...] + e2a1[...] + d2v * qa[...],
                         e2b0[...] + e2b1[...] + d2v * qb[...]],
                        axis=1) + b2[...]
    h2 = _lrelu(z)
    g = lax.broadcasted_iota(jnp.int32, (G, 256), 0)
    cmp = bat[...] == g
    mt = p0[...] + p1[...] + jnp.where(cmp, d2r[...], 0.0)
    psum[...] += jnp.dot(mt, h2, preferred_element_type=jnp.float32)
    cnt[:, 0:1] += jnp.sum(cmp.astype(jnp.float32), axis=1, keepdims=True)

    @pl.when(t == nt - 1)
    def _():
        cg = cnt[:, 0:1]
        pooled = psum[...] / jnp.maximum(cg, 1.0)
        h3 = jnp.dot(pooled, w3[...], preferred_element_type=jnp.float32)
        h3 = h3 + jnp.where(cg > 0, b3[...], 0.0)
        z1 = _lrelu(jnp.dot(h3, fw1[...],
                            preferred_element_type=jnp.float32) + fb1[...])
        z2 = _lrelu(jnp.dot(z1, fw2[...],
                            preferred_element_type=jnp.float32) + fb2[...])
        out[...] = jnp.dot(z2, fw3[...],
                           preferred_element_type=jnp.float32) + fb3[...]


def _make_t3():
    f32 = jnp.float32
    R = 256
    grid = (NP // R,)
    return pl.pallas_call(
        _t3_body,
        grid=grid,
        in_specs=[
            pl.BlockSpec((R, 128), lambda t: (t, 0)),
            pl.BlockSpec((R, 128), lambda t: (t, 0)),
            pl.BlockSpec((R, 128), lambda t: (t, 0)),
            pl.BlockSpec((R, 128), lambda t: (t, 0)),
            pl.BlockSpec((R, 128), lambda t: (t, 0)),
            pl.BlockSpec((R, 128), lambda t: (t, 0)),
            pl.BlockSpec((R, 1), lambda t: (t, 0)),
            pl.BlockSpec((1, 256), lambda t: (0, 0)),
            pl.BlockSpec((1, R), lambda t: (0, t)),
            pl.BlockSpec((1, R), lambda t: (0, t)),
            pl.BlockSpec((G, R), lambda t: (0, t)),
            pl.BlockSpec((G, R), lambda t: (0, t)),
            pl.BlockSpec((256, 256), lambda t: (0, 0)),
            pl.BlockSpec((1, 256), lambda t: (0, 0)),
            pl.BlockSpec((256, 128), lambda t: (0, 0)),
            pl.BlockSpec((1, 128), lambda t: (0, 0)),
            pl.BlockSpec((128, 64), lambda t: (0, 0)),
            pl.BlockSpec((1, 64), lambda t: (0, 0)),
            pl.BlockSpec((64, C), lambda t: (0, 0)),
            pl.BlockSpec((1, C), lambda t: (0, 0)),
        ],
        out_specs=pl.BlockSpec((G, C), lambda t: (0, 0)),
        out_shape=jax.ShapeDtypeStruct((G, C), f32),
        scratch_shapes=[
            pltpu.VMEM((G, 256), f32),
            pltpu.VMEM((G, 128), f32),
        ],
    )


_prep = _make_prep()
_prop1 = _make_prop1()
_prop2 = _make_prop2()
_t2 = _make_t2()
_t3 = _make_t3()


def kernel(x, edge_index, edge_weight, batch,
           W1, b1, W2, b2, W3, b3, FW1, Fb1, FW2, Fb2, FW3, Fb3):
    f32 = jnp.float32
    src = edge_index[0]
    dst = edge_index[1]
    x_pad = jnp.pad(x, ((0, NP - N), (0, 0)))
    batch_pad = jnp.pad(batch, (0, NP - N), constant_values=-1)

    pad_idx = (N + jnp.arange(E_PAD - E, dtype=jnp.int32) % (NP - N))
    src_p = jnp.concatenate([src, pad_idx])
    dst_p = jnp.concatenate([dst, pad_idx])
    ew_p = jnp.pad(edge_weight, (0, E_PAD - E))
    batch_sc = jnp.pad(batch, (0, NP - N))

    dst128 = dst_p.reshape(2560, 128)
    ew128 = ew_p.reshape(2560, 128)
    src128 = src_p.reshape(2560, 128)

    d2, c2d, p0, p1 = _prep(dst128, ew128, src128, batch_sc)

    sd3 = jnp.stack(
        [src_p.reshape(ERR, CH64), dst_p.reshape(ERR, CH64),
         lax.bitcast_convert_type(c2d.reshape(-1), jnp.int32).reshape(
             ERR, CH64)], axis=1)

    e1p0, e1p1 = _prop1(x_pad, sd3)

    d2c = d2.reshape(NP, 1)
    qa, qb = _t2(e1p0, e1p1, x_pad, d2c, W1, b1.reshape(1, H), W2)

    e2a0, e2a1, e2b0, e2b1 = _prop2(qa, qb, sd3)

    out = _t3(e2a0, e2a1, e2b0, e2b1, qa, qb, d2c, b2.reshape(1, H),
              batch_pad.reshape(1, NP), d2.reshape(1, NP),
              p0.reshape(G, NP), p1.reshape(G, NP),
              W3, b3.reshape(1, H), FW1, Fb1.reshape(1, H // 2),
              FW2, Fb2.reshape(1, H // 4), FW3, Fb3.reshape(1, C))
    return out
